# Initial kernel scaffold; baseline (speedup 1.0000x reference)
import jax
import jax.numpy as jnp
from jax.experimental import pallas as pl


def kernel(x_drug, x_target, cell_features, drug1_id, drug2_id, edge_index_dd, edge_index_dt, edge_index_td, edge_index_tt, Ws_dd, Wd_dd, as_dd, ad_dd, b_dd, Ws_dt, Wd_dt, as_dt, ad_dt, b_dt, Ws_td, Wd_td, as_td, ad_td, b_td, Ws_tt, Wd_tt, as_tt, ad_tt, b_tt, r1_W1, r1_b1, r1_W2, r1_b2, r1_W3, r1_b3, r2_W1, r2_b1, r2_W2, r2_b2, r2_W3, r2_b3, cls_W, cls_b):
    raise NotImplementedError("write your pallas kernel here")


# speedup vs baseline: 2.6226x; 2.6226x over previous
import functools

import jax
import jax.numpy as jnp
from jax import lax
from jax.experimental import pallas as pl
from jax.experimental.pallas import tpu as pltpu
from jax.experimental.pallas import tpu_sc as plsc

HIGH = jax.lax.Precision.HIGHEST
ND = 10000
NW = 32
LANES = 16

def _mesh():
    return plsc.VectorSubcoreMesh(core_axis_name="c", subcore_axis_name="s")


def _wid():
    return lax.axis_index("s") * 2 + lax.axis_index("c")


def _k1_body(rels, args):
    n_in = 4 * len(rels)
    n_out = 2 * len(rels)
    ins = args[:n_in]
    outs = args[n_in:n_in + n_out]
    idx_s, idx_d, s_v, d_v, ex_v, psum_v = args[n_in + n_out:]
    wid = _wid()
    for r, (ept, e_real) in enumerate(rels):
        src_h, dst_h, s_h, d_h = ins[4 * r:4 * r + 4]
        ex_o, part_o = outs[2 * r:2 * r + 2]
        base = wid * ept
        pltpu.sync_copy(src_h.at[pl.ds(base, ept)], idx_s.at[pl.ds(0, ept)])
        pltpu.sync_copy(dst_h.at[pl.ds(base, ept)], idx_d.at[pl.ds(0, ept)])
        pltpu.sync_copy(s_h, s_v)
        pltpu.sync_copy(d_h, d_v)

        def zero_body(i):
            psum_v[pl.ds(i * LANES, LANES)] = jnp.zeros((LANES,), jnp.float32)
        lax.fori_loop(0, ND // LANES, lambda i, c: (zero_body(i), c)[1], 0,
                      unroll=4)

        def edge_body(k):
            vis = idx_s[pl.ds(k * LANES, LANES)]
            vid = idx_d[pl.ds(k * LANES, LANES)]
            vs = plsc.load_gather(s_v, [vis])
            vd = plsc.load_gather(d_v, [vid])
            e = vs + vd
            e = jnp.where(e >= 0.0, e, 0.2 * e)
            ex = jnp.exp(e)
            gidx = base + k * LANES + lax.iota(jnp.int32, LANES)
            ex = jnp.where(gidx < e_real, ex, 0.0)
            ex_v[pl.ds(k * LANES, LANES)] = ex
            plsc.addupdate_scatter(psum_v, [vid], ex)
        lax.fori_loop(0, ept // LANES, lambda k, c: (edge_body(k), c)[1], 0,
                      unroll=2)

        pltpu.sync_copy(ex_v.at[pl.ds(0, ept)], ex_o.at[pl.ds(base, ept)])
        pltpu.sync_copy(psum_v, part_o.at[wid])


def _make_k1(epts):
    max_ept = max(e for e, _ in epts)
    out_type = []
    for ept, _ in epts:
        out_type.append(jax.ShapeDtypeStruct((NW * ept,), jnp.float32))
        out_type.append(jax.ShapeDtypeStruct((NW, ND), jnp.float32))
    scratch = [
        pltpu.VMEM((max_ept,), jnp.int32),
        pltpu.VMEM((max_ept,), jnp.int32),
        pltpu.VMEM((ND,), jnp.float32),
        pltpu.VMEM((ND,), jnp.float32),
        pltpu.VMEM((max_ept,), jnp.float32),
        pltpu.VMEM((ND,), jnp.float32),
    ]

    @functools.partial(pl.kernel, mesh=_mesh(), out_type=tuple(out_type),
                       scratch_types=tuple(scratch),
                       compiler_params=pltpu.CompilerParams(
                           needs_layout_passes=False))
    def k1(*args):
        _k1_body(epts, args)

    return k1


G = 128
RPT = 632
RPT_LAST = ND - 15 * RPT


def _k2_body(ept, wc, nchunk, args):
    groups = ept // G
    xs_list = args[:nchunk]
    src_h, dst_h, dst2_h, ex_h, sum_h, acc_o = args[nchunk:nchunk + 6]
    (src_v, dst_v, dst2_v, ex_v, sum_v, rowbuf, acc_sh, sem) = \
        args[nchunk + 6:]
    cid = lax.axis_index("c")
    sid = lax.axis_index("s")
    wid = _wid()
    base = wid * ept
    pltpu.sync_copy(src_h.at[pl.ds(base, ept)], src_v)
    pltpu.sync_copy(dst_h.at[pl.ds(base, ept)], dst_v)
    pltpu.sync_copy(dst2_h.at[pl.ds(wid * groups, groups)], dst2_v)
    pltpu.sync_copy(ex_h.at[pl.ds(base, ept)], ex_v)
    pltpu.sync_copy(sum_h, sum_v)

    def alpha_body(k):
        vid = dst_v[pl.ds(k * LANES, LANES)]
        vsum = plsc.load_gather(sum_v, [vid])
        vex = ex_v[pl.ds(k * LANES, LANES)]
        ex_v[pl.ds(k * LANES, LANES)] = jnp.where(
            vex > 0.0, vex / vsum, 0.0)
    lax.fori_loop(0, ept // LANES, lambda k, c: (alpha_body(k), c)[1], 0,
                  unroll=2)

    def zrow(r):
        for k in range(wc // LANES):
            rowbuf[r, pl.ds(k * LANES, LANES)] = jnp.zeros((LANES,),
                                                           jnp.float32)
    lax.fori_loop(0, G, lambda r, c: (zrow(r), c)[1], 0)

    for c in range(nchunk):
        row0 = sid * RPT

        def zero_rows(nrows):
            for j in range(nrows // G):
                pltpu.sync_copy(rowbuf,
                                acc_sh.at[pl.ds(row0 + j * G, G)])
            rem = nrows - (nrows // G) * G
            if rem:
                pltpu.sync_copy(rowbuf.at[pl.ds(0, rem)],
                                acc_sh.at[pl.ds(row0 + (nrows // G) * G,
                                                rem)])
        @pl.when(sid < 15)
        def _():
            zero_rows(RPT)
        @pl.when(sid == 15)
        def _():
            zero_rows(RPT_LAST)
        plsc.subcore_barrier()

        def group_body(g):
            pltpu.async_copy(xs_list[c].at[src_v.at[pl.ds(g * G, G)]],
                             rowbuf, sem).wait()

            def scale_16rows(rr):
                va = ex_v[pl.ds(g * G + rr * LANES, LANES)]
                for i in range(LANES):
                    r = rr * LANES + i
                    a = va[i]
                    for k in range(wc // LANES):
                        rowbuf[r, pl.ds(k * LANES, LANES)] = (
                            rowbuf[r, pl.ds(k * LANES, LANES)] * a)
            lax.fori_loop(0, G // LANES,
                          lambda rr, cc: (scale_16rows(rr), cc)[1], 0)
            pltpu.sync_copy(rowbuf, acc_sh.at[dst2_v.at[g]], add=True)
        lax.fori_loop(0, groups, lambda g, cc: (group_body(g), cc)[1], 0)
        plsc.subcore_barrier()
        @pl.when(sid < 15)
        def _():
            pltpu.sync_copy(acc_sh.at[pl.ds(row0, RPT)],
                            acc_o.at[c, cid, pl.ds(row0, RPT)])
        @pl.when(sid == 15)
        def _():
            pltpu.sync_copy(acc_sh.at[pl.ds(15 * RPT, RPT_LAST)],
                            acc_o.at[c, cid, pl.ds(15 * RPT, RPT_LAST)])
        plsc.subcore_barrier()
        if c + 1 < nchunk:
            def zrow2(r):
                for k in range(wc // LANES):
                    rowbuf[r, pl.ds(k * LANES, LANES)] = jnp.zeros(
                        (LANES,), jnp.float32)
            lax.fori_loop(0, G, lambda r, cc: (zrow2(r), cc)[1], 0)


def _make_k2(ept, wc, nchunk):
    groups = ept // G
    out_type = jax.ShapeDtypeStruct((nchunk, 2, ND, wc), jnp.float32)
    scratch = [
        pltpu.VMEM((ept,), jnp.int32),
        pltpu.VMEM((ept,), jnp.int32),
        pltpu.VMEM((groups, G), jnp.int32),
        pltpu.VMEM((ept,), jnp.float32),
        pltpu.VMEM((ND,), jnp.float32),
        pltpu.VMEM((G, wc), jnp.float32),
        pltpu.VMEM_SHARED((ND, wc), jnp.float32),
        pltpu.SemaphoreType.DMA,
    ]

    @functools.partial(pl.kernel, mesh=_mesh(), out_type=out_type,
                       scratch_types=tuple(scratch),
                       compiler_params=pltpu.CompilerParams(
                           needs_layout_passes=False))
    def k2(*args):
        _k2_body(ept, wc, nchunk, args)

    return k2


def _make_pair_gather(b, dmodel):
    bpw = b // NW

    @functools.partial(
        pl.kernel, mesh=_mesh(),
        compiler_params=pltpu.CompilerParams(needs_layout_passes=False),
        out_type=jax.ShapeDtypeStruct((b, dmodel), jnp.float32),
        scratch_types=(
            pltpu.VMEM((bpw,), jnp.int32),
            pltpu.VMEM((bpw, dmodel), jnp.float32),
            pltpu.SemaphoreType.DMA,
        ))
    def kg(table_h, idx_h, out_h, idx_v, rows_v, sem):
        base = _wid() * bpw
        pltpu.sync_copy(idx_h.at[pl.ds(base, bpw)], idx_v)
        pltpu.async_copy(table_h.at[idx_v], rows_v, sem).wait()
        pltpu.sync_copy(rows_v, out_h.at[pl.ds(base, bpw)])

    return kg


def _dot(a, b):
    return jnp.dot(a, b, precision=HIGH, preferred_element_type=jnp.float32)


def _scores_call(x, ws):
    nv = len(ws)
    m, kp = x.shape

    def body(x_ref, *rest):
        w_refs = rest[:2 * nv]
        out_ref = rest[2 * nv]
        cols = [_dot(w_refs[2 * i][...], w_refs[2 * i + 1][...])
                for i in range(nv)]
        p = jnp.concatenate(cols, axis=1)
        out_ref[...] = _dot(x_ref[...], p)

    flat = []
    in_specs = [pl.BlockSpec((_BM, kp), lambda i: (i, 0))]
    for w, a in ws:
        flat += [w, a]
        in_specs.append(pl.BlockSpec(w.shape, lambda i: (0, 0)))
        in_specs.append(pl.BlockSpec(a.shape, lambda i: (0, 0)))
    return pl.pallas_call(
        body,
        grid=((m + _BM - 1) // _BM,),
        in_specs=in_specs,
        out_specs=pl.BlockSpec((_BM, nv), lambda i: (i, 0)),
        out_shape=jax.ShapeDtypeStruct((m, nv), jnp.float32),
    )(x, *flat)


def _materialize2d(x):
    def body(x_ref, o_ref):
        o_ref[...] = x_ref[...]
    return pl.pallas_call(
        body, out_shape=jax.ShapeDtypeStruct(x.shape, x.dtype))(x)


def _sum_partials(parts):
    def body(p_ref, o_ref):
        o_ref[...] = jnp.sum(p_ref[...], axis=1)
    return pl.pallas_call(
        body,
        out_shape=jax.ShapeDtypeStruct((parts.shape[0], ND), jnp.float32),
    )(parts)


_BM = 256


def _gat_out_call(acc_list, w_list, b1, b2, n_rows):
    nc = len(acc_list)
    grid = (n_rows + _BM - 1) // _BM

    def body(*refs):
        a_refs = refs[:nc]
        w_refs = refs[nc:2 * nc]
        b1_ref, b2_ref = refs[2 * nc:2 * nc + 2]
        o_ref = refs[2 * nc + 2]
        acc = None
        for i in range(nc):
            a = a_refs[i][0] + a_refs[i][1]
            t = _dot(a, w_refs[i][...])
            acc = t if acc is None else acc + t
        acc = acc + b1_ref[...] + b2_ref[...]
        o_ref[...] = jnp.maximum(acc, 0.0)

    in_specs = []
    for a in acc_list:
        wc = a.shape[-1]
        in_specs.append(pl.BlockSpec((2, _BM, wc), lambda m: (0, m, 0)))
    for w in w_list:
        in_specs.append(pl.BlockSpec(w.shape, lambda m: (0, 0)))
    in_specs.append(pl.BlockSpec((1, 768), lambda m: (0, 0)))
    in_specs.append(pl.BlockSpec((1, 768), lambda m: (0, 0)))
    return pl.pallas_call(
        body,
        grid=(grid,),
        in_specs=in_specs,
        out_specs=pl.BlockSpec((_BM, 768), lambda m: (m, 0)),
        out_shape=jax.ShapeDtypeStruct((n_rows, 768), jnp.float32),
    )(*acc_list, *w_list, b1.reshape(1, -1), b2.reshape(1, -1))


def _mlp_layer(x, w, b, l2norm=False, relu=True, n_block=None):
    m, k = x.shape
    n = w.shape[1]

    def body(x_ref, w_ref, b_ref, o_ref):
        xv = x_ref[...]
        if l2norm:
            nrm = jnp.sqrt(jnp.sum(xv * xv, axis=1, keepdims=True))
            xv = xv / jnp.maximum(nrm, 1e-12)
        y = _dot(xv, w_ref[...]) + b_ref[...]
        if relu:
            y = jnp.maximum(y, 0.0)
        o_ref[...] = y

    if n_block is None:
        return pl.pallas_call(
            body, out_shape=jax.ShapeDtypeStruct((m, n), jnp.float32),
        )(x, w, b.reshape(1, -1))
    return pl.pallas_call(
        body,
        grid=(n // n_block,),
        in_specs=[
            pl.BlockSpec((m, k), lambda j: (0, 0)),
            pl.BlockSpec((k, n_block), lambda j: (0, j)),
            pl.BlockSpec((1, n_block), lambda j: (0, j)),
        ],
        out_specs=pl.BlockSpec((m, n_block), lambda j: (0, j)),
        out_shape=jax.ShapeDtypeStruct((m, n), jnp.float32),
    )(x, w, b.reshape(1, -1))


def _cls_softmax(h, w, b):
    def body(h_ref, w_ref, b_ref, o_ref):
        logits = _dot(h_ref[...], w_ref[...]) + b_ref[...]
        mx = jnp.max(logits, axis=1, keepdims=True)
        e = jnp.exp(logits - mx)
        o_ref[...] = e / jnp.sum(e, axis=1, keepdims=True)
    return pl.pallas_call(
        body, out_shape=jax.ShapeDtypeStruct((h.shape[0], w.shape[1]),
                                             jnp.float32),
    )(h, w, b.reshape(1, -1))


def _pad_edges(ei, self_loops, n_nodes):
    src, dst = ei[0], ei[1]
    if self_loops:
        loop = jnp.arange(n_nodes, dtype=ei.dtype)
        src = jnp.concatenate([src, loop])
        dst = jnp.concatenate([dst, loop])
    e = src.shape[0]
    ept = -(-e // (NW * 1024)) * 1024
    epad = NW * ept
    npad = epad - e
    src = jnp.concatenate([src, jnp.zeros((npad,), ei.dtype)])
    dst = jnp.concatenate([dst, jnp.zeros((npad,), ei.dtype)])
    return src, dst, ept, e


def _pad2(x, rows, cols):
    return jnp.pad(x, ((0, rows - x.shape[0]), (0, cols - x.shape[1])))


def kernel(x_drug, x_target, cell_features, drug1_id, drug2_id,
           edge_index_dd, edge_index_dt, edge_index_td, edge_index_tt,
           Ws_dd, Wd_dd, as_dd, ad_dd, b_dd,
           Ws_dt, Wd_dt, as_dt, ad_dt, b_dt,
           Ws_td, Wd_td, as_td, ad_td, b_td,
           Ws_tt, Wd_tt, as_tt, ad_tt, b_tt,
           r1_W1, r1_b1, r1_W2, r1_b2, r1_W3, r1_b3,
           r2_W1, r2_b1, r2_W2, r2_b2, r2_W3, r2_b3,
           cls_W, cls_b):
    f32 = jnp.float32

    def col_chunks(x, w):
        out = []
        for c0 in range(0, x.shape[1], w):
            blk = x[:, c0:c0 + w]
            if blk.shape[1] < w:
                blk = jnp.pad(blk, ((0, 0), (0, w - blk.shape[1])))
            out.append(blk)
        return out

    def row_chunks(wmat, w):
        out = []
        for c0 in range(0, wmat.shape[0], w):
            blk = wmat[c0:c0 + w]
            if blk.shape[0] < w:
                blk = jnp.pad(blk, ((0, w - blk.shape[0]), (0, 0)))
            out.append(blk)
        return out

    xd_c = col_chunks(x_drug, 128)
    xt_c = col_chunks(x_target, 128)
    Ws_dd_c = row_chunks(Ws_dd, 128)
    Ws_dt_c = row_chunks(Ws_dt, 128)
    Ws_td_c = row_chunks(Ws_td, 128)
    Ws_tt_c = row_chunks(Ws_tt, 128)
    xd_p = _pad2(x_drug, ND, 208)
    Ws_dd_p = _pad2(Ws_dd, 208, 768)
    Ws_dt_p = _pad2(Ws_dt, 208, 768)
    Wd_dd_p = _pad2(Wd_dd, 208, 768)
    Wd_td_p = _pad2(Wd_td, 208, 768)
    xt_p = _pad2(x_target, ND, 576)
    Ws_td_p = _pad2(Ws_td, 576, 768)
    Ws_tt_p = _pad2(Ws_tt, 576, 768)
    Wd_dt_p = _pad2(Wd_dt, 576, 768)
    Wd_tt_p = _pad2(Wd_tt, 576, 768)

    sc_d = _scores_call(xd_p, [
        (Ws_dd_p, as_dd.reshape(-1, 1)), (Wd_dd_p, ad_dd.reshape(-1, 1)),
        (Ws_dt_p, as_dt.reshape(-1, 1)), (Wd_td_p, ad_td.reshape(-1, 1))])
    sc_t = _scores_call(xt_p, [
        (Ws_td_p, as_td.reshape(-1, 1)), (Wd_dt_p, ad_dt.reshape(-1, 1)),
        (Ws_tt_p, as_tt.reshape(-1, 1)), (Wd_tt_p, ad_tt.reshape(-1, 1))])
    s_dd, d_dd = sc_d[:, 0], sc_d[:, 1]
    s_dt, d_td = sc_d[:, 2], sc_d[:, 3]
    s_td, d_dt = sc_t[:, 0], sc_t[:, 1]
    s_tt, d_tt = sc_t[:, 2], sc_t[:, 3]

    src_dd, dst_dd, ept_dd, er_dd = _pad_edges(edge_index_dd, True, ND)
    src_dt, dst_dt, ept_dt, er_dt = _pad_edges(edge_index_dt, False, ND)
    src_td, dst_td, ept_td, er_td = _pad_edges(edge_index_td, False, ND)
    src_tt, dst_tt, ept_tt, er_tt = _pad_edges(edge_index_tt, True, ND)

    k1 = _make_k1(((ept_dd, er_dd), (ept_dt, er_dt),
                   (ept_td, er_td), (ept_tt, er_tt)))
    (ex_dd, p_dd, ex_dt, p_dt, ex_td, p_td, ex_tt, p_tt) = k1(
        src_dd, dst_dd, s_dd, d_dd,
        src_dt, dst_dt, s_dt, d_dt,
        src_td, dst_td, s_td, d_td,
        src_tt, dst_tt, s_tt, d_tt)
    sums = _sum_partials(jnp.stack([p_dd, p_dt, p_td, p_tt]))

    def run_k2(src, dst, ept, ex, ssum, xs_list):
        dst2 = _materialize2d(dst.reshape(-1, G))
        k2 = _make_k2(ept, 128, len(xs_list))
        return k2(*xs_list, src, dst, dst2, ex, ssum)

    acc_dd = run_k2(src_dd, dst_dd, ept_dd, ex_dd, sums[0], xd_c)
    acc_dt = run_k2(src_dt, dst_dt, ept_dt, ex_dt, sums[1], xd_c)
    acc_td = run_k2(src_td, dst_td, ept_td, ex_td, sums[2], xt_c)
    acc_tt = run_k2(src_tt, dst_tt, ept_tt, ex_tt, sums[3], xt_c)

    xd = _gat_out_call(
        [acc_dd[i] for i in range(2)] + [acc_td[i] for i in range(5)],
        Ws_dd_c + Ws_td_c,
        b_dd, b_td, ND)
    xt = _gat_out_call(
        [acc_dt[i] for i in range(2)] + [acc_tt[i] for i in range(5)],
        Ws_dt_c + Ws_tt_c,
        b_dt, b_tt, ND)

    cell = _mlp_layer(_pad2(cell_features, cell_features.shape[0], 896),
                      _pad2(r1_W1, 896, 2048), r1_b1, l2norm=True,
                      n_block=256)
    cell = _mlp_layer(cell, r1_W2, r1_b2)
    cell = _mlp_layer(cell, r1_W3, r1_b3)

    pair_idx = jnp.concatenate([drug1_id, drug2_id])
    kg = _make_pair_gather(2 * drug1_id.shape[0], 768)
    g = kg(xd, pair_idx)
    bsz = drug1_id.shape[0]
    h = jnp.concatenate([g[:bsz], g[bsz:], cell], axis=1)
    h = _mlp_layer(h, r2_W1, r2_b1, l2norm=True, n_block=256)
    h = _mlp_layer(h, r2_W2, r2_b2)
    h = _mlp_layer(h, r2_W3, r2_b3)
    out = _cls_softmax(h, cls_W, cls_b)
    return (out, xd, xt)
